# Initial kernel scaffold; baseline (speedup 1.0000x reference)
import jax
import jax.numpy as jnp
from jax.experimental import pallas as pl


def kernel(x, weight):
    raise NotImplementedError("write your pallas kernel here")


# speedup vs baseline: 1.1240x; 1.1240x over previous
import functools

import jax
import jax.numpy as jnp
from jax.experimental import pallas as pl
from jax.experimental.pallas import tpu as pltpu

_KSIZE = 7
_PAD = _KSIZE // 2
_LANES = 128


def _build_tap_matrices(weight, W, F, ns, pt):
    q = jnp.arange(_LANES)
    w = jnp.arange(_LANES)
    b_src, c_src = q // W, q % W
    b_out, c_out = w // W, w % W
    wf = weight.reshape(2, _KSIZE, _KSIZE).astype(jnp.float32)
    per_d = []
    for d in range(ns):
        acc = jnp.zeros((2, _LANES, _LANES), jnp.float32)
        for i in range(_KSIZE):
            dr = i - _PAD
            delta = (b_out + dr) // F
            bs = (b_out + dr) % F
            row_ok = (delta == (d - pt)) & (b_src[:, None] == bs[None, :])
            for j in range(_KSIZE):
                dc = j - _PAD
                m = row_ok & (c_src[:, None] == (c_out + dc)[None, :])
                acc = acc + wf[:, i, j][:, None, None] * m[None].astype(jnp.float32)
        per_d.append(acc)
    return jnp.stack(per_d, axis=1)


def _fused_body(x_ref, t_ref, o_ref, pad_ref, *, c_total, hf, ns, out_dtype):
    pt = (ns - 1) // 2
    x = x_ref[0]
    mean = jnp.sum(x, axis=0) * (1.0 / float(c_total))
    mx = jnp.max(x, axis=0)

    zrow = jnp.zeros((pt, _LANES), jnp.float32)
    for ci in range(2):
        pad_ref[ci, :pt, :] = zrow
        pad_ref[ci, pt + hf:, :] = zrow
    pad_ref[0, pt:pt + hf, :] = mean
    pad_ref[1, pt:pt + hf, :] = mx

    acc = jnp.zeros((hf, _LANES), jnp.float32)
    for ci in range(2):
        for d in range(ns):
            r = pad_ref[ci, d:d + hf, :]
            acc = acc + jnp.dot(r, t_ref[ci, d],
                                preferred_element_type=jnp.float32)
    o_ref[0, 0] = jax.nn.sigmoid(acc).astype(out_dtype)


def kernel(x, weight):
    N, C, H, W = x.shape
    assert weight.shape == (1, 2, _KSIZE, _KSIZE)
    assert _LANES % W == 0, "W must divide 128"
    F = _LANES // W
    assert H % F == 0, "H must be divisible by the fold factor"
    hf = H // F
    assert hf % 8 == 0, "folded height must be sublane-aligned"

    pt = (_PAD + F - 1) // F
    pb = (F - 1 + _PAD) // F
    assert pt == pb
    ns = pt + pb + 1

    t_mats = _build_tap_matrices(weight, W, F, ns, pt)
    xf = x.reshape(N, C, hf, _LANES)

    body = functools.partial(_fused_body, c_total=C, hf=hf, ns=ns,
                             out_dtype=x.dtype)
    block_bytes = C * hf * _LANES * jnp.dtype(x.dtype).itemsize
    vmem_limit = int(min(2 * block_bytes + (8 << 20), 56 << 20))

    out = pl.pallas_call(
        body,
        out_shape=jax.ShapeDtypeStruct((N, 1, hf, _LANES), x.dtype),
        grid=(N,),
        in_specs=[
            pl.BlockSpec((1, C, hf, _LANES), lambda n: (n, 0, 0, 0)),
            pl.BlockSpec((2, ns, _LANES, _LANES), lambda n: (0, 0, 0, 0)),
        ],
        out_specs=pl.BlockSpec((1, 1, hf, _LANES), lambda n: (n, 0, 0, 0)),
        scratch_shapes=[pltpu.VMEM((2, hf + ns - 1, _LANES), jnp.float32)],
        compiler_params=pltpu.CompilerParams(
            dimension_semantics=("parallel",),
            vmem_limit_bytes=vmem_limit),
    )(xf, t_mats)
    return out.reshape(N, 1, H, W)
